# baseline (speedup 1.0000x reference)
import functools

import jax
import jax.numpy as jnp
from jax import lax
from jax.experimental import pallas as pl
from jax.experimental.pallas import tpu as pltpu
from jax.experimental.pallas import tpu_sc as plsc

MAX_SEQ_LEN = 8192
EMBED_DIM = 1024

_NUM_CORES = 2
_NUM_SUBCORES = 16
_NUM_WORKERS = _NUM_CORES * _NUM_SUBCORES
_ROWS_PER_WORKER = MAX_SEQ_LEN // _NUM_WORKERS
_CHUNK_ROWS = 8
_NUM_CHUNKS = _ROWS_PER_WORKER // _CHUNK_ROWS
_NBUF = 14

_MESH = plsc.VectorSubcoreMesh(core_axis_name="c", subcore_axis_name="s")


@functools.partial(
    pl.kernel,
    mesh=_MESH,
    out_type=jax.ShapeDtypeStruct((MAX_SEQ_LEN, EMBED_DIM), jnp.float32),
    scratch_types=[
        pltpu.VMEM((_NBUF, _CHUNK_ROWS, EMBED_DIM), jnp.float32),
    ]
    + [pltpu.SemaphoreType.DMA] * (2 * _NBUF),
)
def _pos_embed_copy(table_hbm, out_hbm, buf, *sems):
    wid = lax.axis_index("s") * _NUM_CORES + lax.axis_index("c")
    base = wid * _ROWS_PER_WORKER
    in_sems = list(sems[:_NBUF])
    out_sems = list(sems[_NBUF:])

    def start_in(i):
        slot = i % _NBUF
        return pltpu.async_copy(
            table_hbm.at[pl.ds(base + i * _CHUNK_ROWS, _CHUNK_ROWS), :],
            buf.at[slot],
            in_sems[slot],
        )

    def start_out(i):
        slot = i % _NBUF
        return pltpu.async_copy(
            buf.at[slot],
            out_hbm.at[pl.ds(base + i * _CHUNK_ROWS, _CHUNK_ROWS), :],
            out_sems[slot],
        )

    in_dma = [None] * _NUM_CHUNKS
    out_dma = [None] * _NUM_CHUNKS
    for i in range(_NBUF - 1):
        in_dma[i] = start_in(i)
    for i in range(_NUM_CHUNKS):
        in_dma[i].wait()
        out_dma[i] = start_out(i)
        nxt = i + _NBUF - 1
        if nxt < _NUM_CHUNKS:
            if i >= 1:
                out_dma[i - 1].wait()
            in_dma[nxt] = start_in(nxt)
    for i in range(max(0, _NUM_CHUNKS - _NBUF), _NUM_CHUNKS):
        if out_dma[i] is not None:
            out_dma[i].wait()


def kernel(x, pos_embed):
    del x
    return _pos_embed_copy(pos_embed)[None]


# speedup vs baseline: 1.0219x; 1.0219x over previous
import functools

import jax
import jax.numpy as jnp
from jax import lax
from jax.experimental import pallas as pl
from jax.experimental.pallas import tpu as pltpu
from jax.experimental.pallas import tpu_sc as plsc

MAX_SEQ_LEN = 8192
EMBED_DIM = 1024

_NUM_CORES = 2
_ROWS_PER_CORE = MAX_SEQ_LEN // _NUM_CORES
_CHUNK_ROWS = 256
_NUM_CHUNKS = _ROWS_PER_CORE // _CHUNK_ROWS
_NBUF = 4

_MESH = plsc.ScalarSubcoreMesh(axis_name="c", num_cores=_NUM_CORES)


@functools.partial(
    pl.kernel,
    mesh=_MESH,
    out_type=jax.ShapeDtypeStruct((MAX_SEQ_LEN, EMBED_DIM), jnp.float32),
    scratch_types=[
        pltpu.VMEM_SHARED((_NBUF, _CHUNK_ROWS, EMBED_DIM), jnp.float32),
    ]
    + [pltpu.SemaphoreType.DMA] * (2 * _NBUF),
)
def _pos_embed_copy(table_hbm, out_hbm, buf, *sems):
    cid = lax.axis_index("c")
    base = cid * _ROWS_PER_CORE
    in_sems = list(sems[:_NBUF])
    out_sems = list(sems[_NBUF:])

    def start_in(i):
        slot = i % _NBUF
        return pltpu.async_copy(
            table_hbm.at[pl.ds(base + i * _CHUNK_ROWS, _CHUNK_ROWS), :],
            buf.at[slot],
            in_sems[slot],
        )

    def start_out(i):
        slot = i % _NBUF
        return pltpu.async_copy(
            buf.at[slot],
            out_hbm.at[pl.ds(base + i * _CHUNK_ROWS, _CHUNK_ROWS), :],
            out_sems[slot],
        )

    in_dma = [None] * _NUM_CHUNKS
    out_dma = [None] * _NUM_CHUNKS
    for i in range(_NBUF - 1):
        in_dma[i] = start_in(i)
    for i in range(_NUM_CHUNKS):
        in_dma[i].wait()
        out_dma[i] = start_out(i)
        nxt = i + _NBUF - 1
        if nxt < _NUM_CHUNKS:
            if i >= 1:
                out_dma[i - 1].wait()
            in_dma[nxt] = start_in(nxt)
    for i in range(max(0, _NUM_CHUNKS - _NBUF), _NUM_CHUNKS):
        if out_dma[i] is not None:
            out_dma[i].wait()


def kernel(x, pos_embed):
    del x
    return _pos_embed_copy(pos_embed)[None]


# speedup vs baseline: 1.0835x; 1.0603x over previous
import functools

import jax
import jax.numpy as jnp
from jax import lax
from jax.experimental import pallas as pl
from jax.experimental.pallas import tpu as pltpu
from jax.experimental.pallas import tpu_sc as plsc
from jax._src.pallas import mpmd

MAX_SEQ_LEN = 8192
EMBED_DIM = 1024

_NUM_CORES = 2
_NUM_SUBCORES = 16
_NUM_WORKERS = _NUM_CORES * _NUM_SUBCORES

_TEC_ROWS = 4096
_SCS_ROWS = MAX_SEQ_LEN - _TEC_ROWS

_T_ROWS_PER_WORKER = _TEC_ROWS // _NUM_WORKERS
_T_CHUNK = 16
_T_CHUNKS = _T_ROWS_PER_WORKER // _T_CHUNK
_T_NBUF = 6

_S_ROWS_PER_CORE = _SCS_ROWS // _NUM_CORES
_S_CHUNK = 256
_S_CHUNKS = _S_ROWS_PER_CORE // _S_CHUNK
_S_NBUF = 4

_TEC_MESH = plsc.VectorSubcoreMesh(core_axis_name="c", subcore_axis_name="s")
_SCS_MESH = plsc.ScalarSubcoreMesh(axis_name="c", num_cores=_NUM_CORES)


def _pipeline(table_hbm, out_hbm, buf, sems, base, chunk, n_chunks, nbuf):
    in_sems = list(sems[:nbuf])
    out_sems = list(sems[nbuf:])

    def start_in(i):
        slot = i % nbuf
        return pltpu.async_copy(
            table_hbm.at[pl.ds(base + i * chunk, chunk), :],
            buf.at[slot],
            in_sems[slot],
        )

    def start_out(i):
        slot = i % nbuf
        return pltpu.async_copy(
            buf.at[slot],
            out_hbm.at[pl.ds(base + i * chunk, chunk), :],
            out_sems[slot],
        )

    in_dma = [None] * n_chunks
    out_dma = [None] * n_chunks
    for i in range(min(nbuf - 1, n_chunks)):
        in_dma[i] = start_in(i)
    for i in range(n_chunks):
        in_dma[i].wait()
        out_dma[i] = start_out(i)
        nxt = i + nbuf - 1
        if nxt < n_chunks:
            if i >= 1:
                out_dma[i - 1].wait()
            in_dma[nxt] = start_in(nxt)
    for i in range(max(0, n_chunks - nbuf), n_chunks):
        if out_dma[i] is not None:
            out_dma[i].wait()


def _tec_fn(table_hbm, out_hbm):
    def body(buf, *sems):
        wid = lax.axis_index("s") * _NUM_CORES + lax.axis_index("c")
        base = wid * _T_ROWS_PER_WORKER
        _pipeline(table_hbm, out_hbm, buf, sems, base, _T_CHUNK, _T_CHUNKS, _T_NBUF)

    pl.run_scoped(
        body,
        pltpu.VMEM((_T_NBUF, _T_CHUNK, EMBED_DIM), jnp.float32),
        *([pltpu.SemaphoreType.DMA] * (2 * _T_NBUF)),
    )


def _scs_fn(table_hbm, out_hbm):
    def body(buf, *sems):
        cid = lax.axis_index("c")
        base = _TEC_ROWS + cid * _S_ROWS_PER_CORE
        _pipeline(table_hbm, out_hbm, buf, sems, base, _S_CHUNK, _S_CHUNKS, _S_NBUF)

    pl.run_scoped(
        body,
        pltpu.VMEM_SHARED((_S_NBUF, _S_CHUNK, EMBED_DIM), jnp.float32),
        *([pltpu.SemaphoreType.DMA] * (2 * _S_NBUF)),
    )


_copy = mpmd.mpmd_map(
    [(_SCS_MESH, _scs_fn), (_TEC_MESH, _tec_fn)],
    out_types=jax.ShapeDtypeStruct((MAX_SEQ_LEN, EMBED_DIM), jnp.float32),
)


def kernel(x, pos_embed):
    del x
    return _copy(pos_embed)[None]
